# baseline (speedup 1.0000x reference)
import jax
import jax.numpy as jnp
from jax.experimental import pallas as pl
from jax.experimental.pallas import tpu as pltpu

_S = 4
_N = 16
_E = 256
_D = 256
_P = 300

_SPB = 4


def _slice_body(tl_ref, nf_ref, ef_ref, adj_ref, w1_ref, w2_ref, w3_ref,
                beta_ref, out_ref, vsum_ref, vcnt_ref):
    i = pl.program_id(0)
    nsteps = pl.num_programs(0)
    nf_all = nf_ref[...].reshape(_SPB * _N, _D)
    ef_all = ef_ref[...].reshape(_SPB * _E, _D)

    dn_t = (((1,), (1,)), ((), ()))
    A_all = jax.lax.dot_general(nf_all, w1_ref[...], dn_t,
                                preferred_element_type=jnp.float32)
    C_all = jax.lax.dot_general(nf_all, w3_ref[...], dn_t,
                                preferred_element_type=jnp.float32) + beta_ref[...]
    Bm_all = jax.lax.dot_general(ef_all, w2_ref[...], dn_t,
                                 preferred_element_type=jnp.float32)

    ones_p = jnp.ones((1, _P), dtype=jnp.float32)
    sC_all = jax.lax.dot_general(ones_p, C_all * C_all, (((1,), (1,)), ((), ())),
                                 preferred_element_type=jnp.float32)
    sB_all = jax.lax.dot_general(ones_p, Bm_all * Bm_all, (((1,), (1,)), ((), ())),
                                 preferred_element_type=jnp.float32)
    sA_all = jnp.sum(A_all * A_all, axis=1, keepdims=True)

    n_iota = jax.lax.broadcasted_iota(jnp.int32, (_N, _E), 0)
    e_iota = jax.lax.broadcasted_iota(jnp.int32, (_N, _E), 1)
    lo_mask = e_iota < n_iota * _N
    hi_mask = e_iota >= (n_iota + 1) * _N
    m16 = (e_iota % _N) == n_iota

    @pl.when(i == 0)
    def _():
        vsum_ref[...] = jnp.zeros((_N, _E), jnp.float32)
        vcnt_ref[...] = jnp.zeros((_N, _E), jnp.float32)

    vsum = vsum_ref[...]
    vcnt = vcnt_ref[...]
    for s in range(_SPB):
        A = A_all[s * _N:(s + 1) * _N]
        C = C_all[s * _N:(s + 1) * _N]
        Bm = Bm_all[s * _E:(s + 1) * _E]
        sC = sC_all[:, s * _N:(s + 1) * _N]
        sB = sB_all[:, s * _E:(s + 1) * _E]
        sA = sA_all[s * _N:(s + 1) * _N]

        CAstack = jnp.concatenate([C, A], axis=0)
        CAB = jax.lax.dot_general(CAstack, Bm, (((1,), (1,)), ((), ())),
                                  preferred_element_type=jnp.float32)
        CB = CAB[:_N]
        BA = CAB[_N:]
        CA = jax.lax.dot_general(A, C, (((1,), (1,)), ((), ())),
                                 preferred_element_type=jnp.float32)

        T = sC - 2.0 * CA

        band_cb = jnp.sum(jnp.where(m16, CB, 0.0), axis=0,
                          keepdims=True)
        band_t = jnp.concatenate([T] * (_E // _N), axis=1)
        band = band_t - 2.0 * band_cb
        lo = T[:, 0:1] - 2.0 * CB[0:1, :]
        hi = T[:, _N - 1:_N] - 2.0 * CB[_N - 1:_N, :]
        acc = jnp.where(lo_mask, lo, jnp.where(hi_mask, hi, band))

        per_rel = (acc + sB + sA + 2.0 * BA) * (1.0 / float(_P))

        tl = tl_ref[i * (_SPB // _S) + s // _S]
        validf = jnp.where(s % _S < tl, 1.0, 0.0)
        mask = (adj_ref[s] == 1) & (per_rel > 0.0)
        vsum = vsum + jnp.where(mask, per_rel, 0.0) * validf
        vcnt = vcnt + mask.astype(jnp.float32) * validf
    vsum_ref[...] = vsum
    vcnt_ref[...] = vcnt

    @pl.when(i == nsteps - 1)
    def _():
        out_ref[...] = (jnp.sum(vsum) / jnp.sum(vcnt)).reshape(1, 1)


def kernel(nodes_feats, edges_feats, adj_mat, true_lengths, W1, b1, W2, b2, W3, b3):
    BS = nodes_feats.shape[0]
    beta = (b3 - b2 - b1).reshape(1, _P)

    grid = (BS // _SPB,)
    out = pl.pallas_call(
        _slice_body,
        grid=grid,
        in_specs=[
            pl.BlockSpec(memory_space=pltpu.SMEM),
            pl.BlockSpec((_SPB, _N, _D), lambda i: (i, 0, 0)),
            pl.BlockSpec((_SPB, _E, _D), lambda i: (i, 0, 0)),
            pl.BlockSpec((_SPB, _N, _E), lambda i: (i, 0, 0)),
            pl.BlockSpec((_P, _D), lambda i: (0, 0)),
            pl.BlockSpec((_P, _D), lambda i: (0, 0)),
            pl.BlockSpec((_P, _D), lambda i: (0, 0)),
            pl.BlockSpec((1, _P), lambda i: (0, 0)),
        ],
        out_specs=pl.BlockSpec((1, 1), lambda i: (0, 0)),
        out_shape=jax.ShapeDtypeStruct((1, 1), jnp.float32),
        scratch_shapes=[pltpu.VMEM((_N, _E), jnp.float32),
                        pltpu.VMEM((_N, _E), jnp.float32)],
        compiler_params=pltpu.CompilerParams(
            dimension_semantics=("arbitrary",)),
    )(true_lengths, nodes_feats, edges_feats, adj_mat, W1, W2, W3, beta)
    return out[0, 0]


# speedup vs baseline: 1.1903x; 1.1903x over previous
import jax
import jax.numpy as jnp
from jax.experimental import pallas as pl
from jax.experimental.pallas import tpu as pltpu

_S = 4
_N = 16
_E = 256
_D = 256
_P = 300

_SPB = 8


def _slice_body(tl_ref, nf_ref, ef_ref, adj_ref, w1_ref, w2_ref, w3_ref,
                beta_ref, out_ref, vsum_ref, vcnt_ref):
    i = pl.program_id(0)
    nsteps = pl.num_programs(0)
    nf_all = nf_ref[...].reshape(_SPB * _N, _D)
    ef_all = ef_ref[...].reshape(_SPB * _E, _D)

    dn_t = (((1,), (1,)), ((), ()))
    A_all = jax.lax.dot_general(nf_all, w1_ref[...], dn_t,
                                preferred_element_type=jnp.float32)
    C_all = jax.lax.dot_general(nf_all, w3_ref[...], dn_t,
                                preferred_element_type=jnp.float32) + beta_ref[...]
    Bm_all = jax.lax.dot_general(ef_all, w2_ref[...], dn_t,
                                 preferred_element_type=jnp.float32)

    ones_p = jnp.ones((1, _P), dtype=jnp.float32)
    sC_all = jax.lax.dot_general(ones_p, C_all * C_all, (((1,), (1,)), ((), ())),
                                 preferred_element_type=jnp.float32)
    sB_all = jax.lax.dot_general(ones_p, Bm_all * Bm_all, (((1,), (1,)), ((), ())),
                                 preferred_element_type=jnp.float32)
    sA_all = jnp.sum(A_all * A_all, axis=1, keepdims=True)
    CA_full = jax.lax.dot_general(A_all, C_all, (((1,), (1,)), ((), ())),
                                  preferred_element_type=jnp.float32)

    n_iota = jax.lax.broadcasted_iota(jnp.int32, (_N, _E), 0)
    e_iota = jax.lax.broadcasted_iota(jnp.int32, (_N, _E), 1)
    lo_mask = e_iota < n_iota * _N
    hi_mask = e_iota >= (n_iota + 1) * _N
    m16 = (e_iota % _N) == n_iota

    @pl.when(i == 0)
    def _():
        vsum_ref[...] = jnp.zeros((_N, _E), jnp.float32)
        vcnt_ref[...] = jnp.zeros((_N, _E), jnp.float32)

    vsum = vsum_ref[...]
    vcnt = vcnt_ref[...]
    for s in range(_SPB):
        A = A_all[s * _N:(s + 1) * _N]
        C = C_all[s * _N:(s + 1) * _N]
        Bm = Bm_all[s * _E:(s + 1) * _E]
        sC = sC_all[:, s * _N:(s + 1) * _N]
        sB = sB_all[:, s * _E:(s + 1) * _E]
        sA = sA_all[s * _N:(s + 1) * _N]

        CAstack = jnp.concatenate([C, A], axis=0)
        CAB = jax.lax.dot_general(CAstack, Bm, (((1,), (1,)), ((), ())),
                                  preferred_element_type=jnp.float32)
        CB = CAB[:_N]
        BA = CAB[_N:]
        CA = CA_full[s * _N:(s + 1) * _N, s * _N:(s + 1) * _N]

        T = sC - 2.0 * CA

        band_cb = jnp.sum(jnp.where(m16, CB, 0.0), axis=0,
                          keepdims=True)
        band_t = jnp.concatenate([T] * (_E // _N), axis=1)
        band = band_t - 2.0 * band_cb
        lo = T[:, 0:1] - 2.0 * CB[0:1, :]
        hi = T[:, _N - 1:_N] - 2.0 * CB[_N - 1:_N, :]
        acc = jnp.where(lo_mask, lo, jnp.where(hi_mask, hi, band))

        per_rel = (acc + sB + sA + 2.0 * BA) * (1.0 / float(_P))

        tl = tl_ref[i * (_SPB // _S) + s // _S]
        validf = jnp.where(s % _S < tl, 1.0, 0.0)
        mask = (adj_ref[s] == 1) & (per_rel > 0.0)
        vsum = vsum + jnp.where(mask, per_rel, 0.0) * validf
        vcnt = vcnt + mask.astype(jnp.float32) * validf
    vsum_ref[...] = vsum
    vcnt_ref[...] = vcnt

    @pl.when(i == nsteps - 1)
    def _():
        out_ref[...] = (jnp.sum(vsum) / jnp.sum(vcnt)).reshape(1, 1)


def kernel(nodes_feats, edges_feats, adj_mat, true_lengths, W1, b1, W2, b2, W3, b3):
    BS = nodes_feats.shape[0]
    beta = (b3 - b2 - b1).reshape(1, _P)

    grid = (BS // _SPB,)
    out = pl.pallas_call(
        _slice_body,
        grid=grid,
        in_specs=[
            pl.BlockSpec(memory_space=pltpu.SMEM),
            pl.BlockSpec((_SPB, _N, _D), lambda i: (i, 0, 0)),
            pl.BlockSpec((_SPB, _E, _D), lambda i: (i, 0, 0)),
            pl.BlockSpec((_SPB, _N, _E), lambda i: (i, 0, 0)),
            pl.BlockSpec((_P, _D), lambda i: (0, 0)),
            pl.BlockSpec((_P, _D), lambda i: (0, 0)),
            pl.BlockSpec((_P, _D), lambda i: (0, 0)),
            pl.BlockSpec((1, _P), lambda i: (0, 0)),
        ],
        out_specs=pl.BlockSpec((1, 1), lambda i: (0, 0)),
        out_shape=jax.ShapeDtypeStruct((1, 1), jnp.float32),
        scratch_shapes=[pltpu.VMEM((_N, _E), jnp.float32),
                        pltpu.VMEM((_N, _E), jnp.float32)],
        compiler_params=pltpu.CompilerParams(
            dimension_semantics=("arbitrary",)),
    )(true_lengths, nodes_feats, edges_feats, adj_mat, W1, W2, W3, beta)
    return out[0, 0]


# speedup vs baseline: 1.1918x; 1.0013x over previous
import jax
import jax.numpy as jnp
from jax.experimental import pallas as pl
from jax.experimental.pallas import tpu as pltpu

_S = 4
_N = 16
_E = 256
_D = 256
_P = 300

_SPB = 8


def _slice_body(tl_ref, nf_ref, ef_ref, adj_ref, w1_ref, w2_ref, w3_ref,
                beta_ref, out_ref, vsum_ref, vcnt_ref):
    i = pl.program_id(0)
    nsteps = pl.num_programs(0)
    nf_all = nf_ref[...].reshape(_SPB * _N, _D)
    ef_all = ef_ref[...].reshape(_SPB * _E, _D)

    dn_t = (((1,), (1,)), ((), ()))
    A_all = jax.lax.dot_general(nf_all, w1_ref[...], dn_t,
                                preferred_element_type=jnp.float32)
    C_all = jax.lax.dot_general(nf_all, w3_ref[...], dn_t,
                                preferred_element_type=jnp.float32) + beta_ref[...]
    Bm_all = jax.lax.dot_general(ef_all, w2_ref[...], dn_t,
                                 preferred_element_type=jnp.float32)

    ones_p = jnp.ones((1, _P), dtype=jnp.float32)
    sC_all = jax.lax.dot_general(ones_p, C_all * C_all, (((1,), (1,)), ((), ())),
                                 preferred_element_type=jnp.float32)
    sB_all = jax.lax.dot_general(ones_p, Bm_all * Bm_all, (((1,), (1,)), ((), ())),
                                 preferred_element_type=jnp.float32)
    sA_all = jnp.sum(A_all * A_all, axis=1, keepdims=True)
    CA_full = jax.lax.dot_general(A_all, C_all, (((1,), (1,)), ((), ())),
                                  preferred_element_type=jnp.float32)

    n_iota = jax.lax.broadcasted_iota(jnp.int32, (_N, _E), 0)
    e_iota = jax.lax.broadcasted_iota(jnp.int32, (_N, _E), 1)
    lo_mask = e_iota < n_iota * _N
    hi_mask = e_iota >= (n_iota + 1) * _N
    m16 = (e_iota % _N) == n_iota

    @pl.when(i == 0)
    def _():
        vsum_ref[...] = jnp.zeros((_N, _E), jnp.float32)
        vcnt_ref[...] = jnp.zeros((_N, _E), jnp.float32)

    vsum = vsum_ref[...]
    vcnt = vcnt_ref[...]
    for s in range(_SPB):
        A = A_all[s * _N:(s + 1) * _N]
        C = C_all[s * _N:(s + 1) * _N]
        Bm = Bm_all[s * _E:(s + 1) * _E]
        sC = sC_all[:, s * _N:(s + 1) * _N]
        sB = sB_all[:, s * _E:(s + 1) * _E]
        sA = sA_all[s * _N:(s + 1) * _N]

        CAstack = jnp.concatenate([C, A], axis=0)
        CAB = jax.lax.dot_general(CAstack, Bm, (((1,), (1,)), ((), ())),
                                  preferred_element_type=jnp.float32)
        CB = CAB[:_N]
        BA = CAB[_N:]
        CA = CA_full[s * _N:(s + 1) * _N, s * _N:(s + 1) * _N]

        T = sC - 2.0 * CA

        band_cb = jnp.sum(jnp.where(m16, CB, 0.0), axis=0,
                          keepdims=True)
        band_t = jnp.concatenate([T] * (_E // _N), axis=1)
        band = band_t - 2.0 * band_cb
        lo = T[:, 0:1] - 2.0 * CB[0:1, :]
        hi = T[:, _N - 1:_N] - 2.0 * CB[_N - 1:_N, :]
        acc = jnp.where(lo_mask, lo, jnp.where(hi_mask, hi, band))

        per_rel = (acc + sB + sA + 2.0 * BA) * (1.0 / float(_P))

        tl = tl_ref[i * (_SPB // _S) + s // _S]
        valid = s % _S < tl
        mask = (adj_ref[s] == 1) & (per_rel > 0.0) & valid
        vsum = vsum + jnp.where(mask, per_rel, 0.0)
        vcnt = vcnt + mask.astype(jnp.float32)
    vsum_ref[...] = vsum
    vcnt_ref[...] = vcnt

    @pl.when(i == nsteps - 1)
    def _():
        out_ref[...] = (jnp.sum(vsum) / jnp.sum(vcnt)).reshape(1, 1)


def kernel(nodes_feats, edges_feats, adj_mat, true_lengths, W1, b1, W2, b2, W3, b3):
    BS = nodes_feats.shape[0]
    beta = (b3 - b2 - b1).reshape(1, _P)

    grid = (BS // _SPB,)
    out = pl.pallas_call(
        _slice_body,
        grid=grid,
        in_specs=[
            pl.BlockSpec(memory_space=pltpu.SMEM),
            pl.BlockSpec((_SPB, _N, _D), lambda i: (i, 0, 0)),
            pl.BlockSpec((_SPB, _E, _D), lambda i: (i, 0, 0)),
            pl.BlockSpec((_SPB, _N, _E), lambda i: (i, 0, 0)),
            pl.BlockSpec((_P, _D), lambda i: (0, 0)),
            pl.BlockSpec((_P, _D), lambda i: (0, 0)),
            pl.BlockSpec((_P, _D), lambda i: (0, 0)),
            pl.BlockSpec((1, _P), lambda i: (0, 0)),
        ],
        out_specs=pl.BlockSpec((1, 1), lambda i: (0, 0)),
        out_shape=jax.ShapeDtypeStruct((1, 1), jnp.float32),
        scratch_shapes=[pltpu.VMEM((_N, _E), jnp.float32),
                        pltpu.VMEM((_N, _E), jnp.float32)],
        compiler_params=pltpu.CompilerParams(
            dimension_semantics=("arbitrary",)),
    )(true_lengths, nodes_feats, edges_feats, adj_mat, W1, W2, W3, beta)
    return out[0, 0]
